# baseline (speedup 1.0000x reference)
import functools

import jax
import jax.numpy as jnp
from jax import lax
from jax.experimental import pallas as pl
from jax.experimental.pallas import tpu as pltpu
from jax.experimental.pallas import tpu_sc as plsc

N_NODES = 10000
N_EDGES = 160000
D_IN = 256
H1, C1 = 8, 8
NCLS = 40

NPAD = 10016
NW = 32
GROUPS = 40
GB = 128
EPAD = NW * GROUPS * GB
ROWS_PER_SUB = NPAD // 16

ACC1_COLS = 72
TBL1_COLS = 96
ACC2_COLS = 48
TBL2_COLS = 64
BCOLS = 16


def _take16(v, idx):
    return lax.gather(
        v, idx[:, None],
        lax.GatherDimensionNumbers(
            offset_dims=(), collapsed_slice_dims=(0,), start_index_map=(0,)),
        slice_sizes=(1,),
        mode=lax.GatherScatterMode.PROMISE_IN_BOUNDS)


def _k1_body(x_ref, wa_ref, m_ref, wb_ref, a_out, b_out):
    xb = x_ref[...]
    a = jnp.dot(xb, wa_ref[...], preferred_element_type=jnp.float32)
    a_out[...] = (a + m_ref[...]).astype(jnp.bfloat16)
    b_out[...] = jnp.dot(xb, wb_ref[...], preferred_element_type=jnp.float32)


def _run_k1(x_pad, wa1, m1, wb1):
    blk = NPAD // 4
    grid = 4
    return pl.pallas_call(
        _k1_body,
        grid=(grid,),
        in_specs=[
            pl.BlockSpec((blk, D_IN), lambda i: (i, 0)),
            pl.BlockSpec((D_IN, TBL1_COLS), lambda i: (0, 0)),
            pl.BlockSpec((1, TBL1_COLS), lambda i: (0, 0)),
            pl.BlockSpec((D_IN, BCOLS), lambda i: (0, 0)),
        ],
        out_specs=[
            pl.BlockSpec((blk, TBL1_COLS), lambda i: (i, 0)),
            pl.BlockSpec((blk, BCOLS), lambda i: (i, 0)),
        ],
        out_shape=[
            jax.ShapeDtypeStruct((NPAD, TBL1_COLS), jnp.bfloat16),
            jax.ShapeDtypeStruct((NPAD, BCOLS), jnp.float32),
        ],
    )(x_pad, wa1, m1, wb1)


def _edge_body1(rows_a, rows_b, ctr, e):
    u0 = rows_a[e, pl.ds(0, 32)].astype(jnp.float32)
    u1 = rows_a[e, pl.ds(32, 32)].astype(jnp.float32)
    u2 = rows_a[e, pl.ds(64, 32)].astype(jnp.float32)
    g0, g1 = u0[0:16], u0[16:32]
    g2, g3 = u1[0:16], u1[16:32]
    asv = u2[0:16]
    a_d = rows_b[e, :]
    al = asv + a_d
    al = jnp.maximum(al, al * jnp.float32(0.2))
    ex = jnp.exp(al)
    iot = lax.iota(jnp.int32, 16)
    half = lax.shift_right_logical(iot, 3)
    low8 = iot < 8
    gs = (g0, g1, g2, g3)
    for k in range(4):
        m = _take16(ex, half + 2 * k)
        ctr[e, pl.ds(16 * k, 16)] = gs[k] * m
    g3s = _take16(g3, jnp.where(low8, iot + 8, 0))
    v = jnp.where(low8, g3s, jnp.float32(1.0))
    mlast = _take16(ex, jnp.where(low8, 7, iot - 8))
    ctr[e, pl.ds(56, 16)] = v * mlast


def _edge_body2(rows_a, rows_b, ctr, e):
    u0 = rows_a[e, pl.ds(0, 32)].astype(jnp.float32)
    u1 = rows_a[e, pl.ds(32, 32)].astype(jnp.float32)
    g0, g1 = u0[0:16], u0[16:32]
    g2 = u1[0:16]
    a_d = rows_b[e, :]
    al = g2 + a_d
    al = jnp.maximum(al, al * jnp.float32(0.2))
    ex = jnp.exp(al)
    m = _take16(ex, jnp.full((16,), 9, jnp.int32))
    gs = (g0, g1, g2)
    for k in range(3):
        ctr[e, pl.ds(16 * k, 16)] = gs[k] * m


def _make_sc_pass(acc_cols, tbl_cols, edge_body, stage_a):
    mesh = plsc.VectorSubcoreMesh(core_axis_name="c", subcore_axis_name="s")

    @functools.partial(
        pl.kernel, mesh=mesh,
        compiler_params=pltpu.CompilerParams(use_tc_tiling_on_sc=False),
        out_type=[jax.ShapeDtypeStruct((NPAD, acc_cols), jnp.float32),
                  jax.ShapeDtypeStruct((NPAD, acc_cols), jnp.float32)],
        scratch_types=[
            pltpu.VMEM_SHARED((NPAD, acc_cols), jnp.float32),
            pltpu.VMEM_SHARED((NPAD if stage_a else 1,
                               tbl_cols), jnp.bfloat16),
            pltpu.VMEM((GROUPS, GB), jnp.int32),
            pltpu.VMEM((GROUPS, GB), jnp.int32),
            pltpu.VMEM((2, GB, tbl_cols), jnp.bfloat16),
            pltpu.VMEM((2, GB, BCOLS), jnp.float32),
            pltpu.VMEM((2, GB, acc_cols), jnp.float32),
            pltpu.SemaphoreType.DMA,
            pltpu.SemaphoreType.DMA,
            pltpu.SemaphoreType.DMA,
            pltpu.SemaphoreType.DMA,
            pltpu.SemaphoreType.DMA,
            pltpu.SemaphoreType.DMA,
        ],
    )
    def sc_pass(a_hbm, b_hbm, src_hbm, dst_hbm, z_hbm, out0_hbm, out1_hbm,
                acc, a_sp, sidx, didx, rows_a, rows_b, ctr,
                sa0, sa1, sb0, sb1, ss0, ss1):
        c = lax.axis_index("c")
        s = lax.axis_index("s")
        w = s * 2 + c
        r0 = s * ROWS_PER_SUB
        sem_a = (sa0, sa1)
        sem_b = (sb0, sb1)
        sem_s = (ss0, ss1)
        rows_slice = pl.ds(r0, ROWS_PER_SUB)
        pltpu.sync_copy(z_hbm, acc.at[rows_slice])
        if stage_a:
            pltpu.sync_copy(a_hbm.at[rows_slice], a_sp.at[rows_slice])
        pltpu.sync_copy(src_hbm.at[w], sidx)
        pltpu.sync_copy(dst_hbm.at[w], didx)
        plsc.subcore_barrier()

        a_tbl = a_sp if stage_a else a_hbm

        def start_gather(g, b):
            pltpu.async_copy(a_tbl.at[sidx.at[g]], rows_a.at[b], sem_a[b])
            pltpu.async_copy(b_hbm.at[didx.at[g]], rows_b.at[b], sem_b[b])

        start_gather(0, 0)
        start_gather(1, 1)

        def gp_body(gp, carry):
            for b in range(2):
                g = 2 * gp + b
                pltpu.make_async_copy(
                    a_tbl.at[sidx.at[g]], rows_a.at[b], sem_a[b]).wait()
                pltpu.make_async_copy(
                    b_hbm.at[didx.at[g]], rows_b.at[b], sem_b[b]).wait()

                @pl.when(gp > 0)
                def _():
                    pltpu.make_async_copy(
                        ctr.at[b], acc.at[didx.at[g]], sem_s[b]).wait()

                @plsc.parallel_loop(0, GB, unroll=16)
                def _(e):
                    edge_body(rows_a.at[b], rows_b.at[b], ctr.at[b], e)

                pltpu.async_copy(ctr.at[b], acc.at[didx.at[g]], sem_s[b],
                                 add=True)

                @pl.when(g + 2 < GROUPS)
                def _():
                    start_gather(g + 2, b)
            return carry
        lax.fori_loop(0, GROUPS // 2, gp_body, 0)

        for b in range(2):
            pltpu.make_async_copy(
                ctr.at[b], acc.at[didx.at[GROUPS - 2 + b]], sem_s[b]).wait()

        plsc.subcore_barrier()

        @pl.when(c == 0)
        def _():
            pltpu.sync_copy(acc.at[rows_slice], out0_hbm.at[rows_slice])

        @pl.when(c == 1)
        def _():
            pltpu.sync_copy(acc.at[rows_slice], out1_hbm.at[rows_slice])

    return sc_pass


def _k2_body(p0_ref, p1_ref, b1_ref, e8_ref, w2a_ref, m2_ref, w2b_ref,
             a_out, b_out):
    acc = p0_ref[...] + p1_ref[...]
    num = acc[:, 0:64]
    den = jnp.dot(acc[:, 64:72], e8_ref[...],
                  preferred_element_type=jnp.float32)
    h1 = num / (den + jnp.float32(1e-16)) + b1_ref[...]
    h1 = jnp.where(h1 > 0, h1, jnp.exp(jnp.minimum(h1, 0.0)) - 1.0)
    a2 = jnp.dot(h1, w2a_ref[...], preferred_element_type=jnp.float32)
    a_out[...] = (a2 + m2_ref[...]).astype(jnp.bfloat16)
    b_out[...] = jnp.dot(h1, w2b_ref[...], preferred_element_type=jnp.float32)


def _run_k2(p1, b1, e8, w2a, m2, w2b):
    blk = NPAD // 4
    grid = 4
    return pl.pallas_call(
        _k2_body,
        grid=(grid,),
        in_specs=[
            pl.BlockSpec((blk, ACC1_COLS), lambda i: (i, 0)),
            pl.BlockSpec((blk, ACC1_COLS), lambda i: (i, 0)),
            pl.BlockSpec((1, 64), lambda i: (0, 0)),
            pl.BlockSpec((8, 64), lambda i: (0, 0)),
            pl.BlockSpec((64, TBL2_COLS), lambda i: (0, 0)),
            pl.BlockSpec((1, TBL2_COLS), lambda i: (0, 0)),
            pl.BlockSpec((64, BCOLS), lambda i: (0, 0)),
        ],
        out_specs=[
            pl.BlockSpec((blk, TBL2_COLS), lambda i: (i, 0)),
            pl.BlockSpec((blk, BCOLS), lambda i: (i, 0)),
        ],
        out_shape=[
            jax.ShapeDtypeStruct((NPAD, TBL2_COLS), jnp.bfloat16),
            jax.ShapeDtypeStruct((NPAD, BCOLS), jnp.float32),
        ],
    )(p1[0], p1[1], b1, e8, w2a, m2, w2b)


def _k3_body(p0_ref, p1_ref, b2_ref, out_ref):
    acc = p0_ref[...] + p1_ref[...]
    num = acc[:, 0:NCLS]
    den = acc[:, NCLS:NCLS + 1]
    logits = num / (den + jnp.float32(1e-16)) + b2_ref[...]
    mx = jnp.max(logits, axis=1, keepdims=True)
    lse = jnp.log(jnp.sum(jnp.exp(logits - mx), axis=1, keepdims=True))
    out_ref[...] = logits - mx - lse


def _run_k3(p2, b2):
    blk = NPAD // 4
    grid = 4
    return pl.pallas_call(
        _k3_body,
        grid=(grid,),
        in_specs=[
            pl.BlockSpec((blk, ACC2_COLS), lambda i: (i, 0)),
            pl.BlockSpec((blk, ACC2_COLS), lambda i: (i, 0)),
            pl.BlockSpec((1, NCLS), lambda i: (0, 0)),
        ],
        out_specs=pl.BlockSpec((blk, NCLS), lambda i: (i, 0)),
        out_shape=jax.ShapeDtypeStruct((NPAD, NCLS), jnp.float32),
    )(p2[0], p2[1], b2)


_sc_pass1 = _make_sc_pass(ACC1_COLS, TBL1_COLS, _edge_body1, stage_a=True)
_sc_pass2 = _make_sc_pass(ACC2_COLS, TBL2_COLS, _edge_body2, stage_a=True)


@jax.jit
def kernel(x, edge_index, W1, att_src1, att_dst1, b1, W2, att_src2, att_dst2, b2):
    f32 = jnp.float32
    ei = edge_index.astype(jnp.int32)
    pad = jnp.full((EPAD - N_EDGES,), N_NODES, jnp.int32)
    src = jnp.concatenate([ei[0], pad]).reshape(NW, GROUPS, GB)
    dst = jnp.concatenate([ei[1], pad]).reshape(NW, GROUPS, GB)

    e8 = jnp.repeat(jnp.eye(H1, dtype=f32), C1, axis=1).reshape(H1, H1 * C1)

    wsrc1 = (W1 * att_src1.reshape(1, H1 * C1)) @ e8.T
    wdst1 = (W1 * att_dst1.reshape(1, H1 * C1)) @ e8.T

    wa1 = jnp.concatenate([W1, wsrc1, jnp.zeros((D_IN, 24), f32)], axis=1)
    m1 = (((jnp.arange(TBL1_COLS) >= 72)
           & (jnp.arange(TBL1_COLS) < 80)).astype(f32)
          .reshape(1, TBL1_COLS))
    wb1 = jnp.concatenate([wdst1, jnp.zeros((D_IN, 8), f32)], axis=1)

    w2a = jnp.concatenate(
        [W2, jnp.zeros((64, 1), f32), (W2 @ att_src2[0])[:, None],
         jnp.zeros((64, 22), f32)], axis=1)
    m2 = (jnp.arange(TBL2_COLS) == 40).astype(f32).reshape(1, TBL2_COLS)
    w2b = jnp.zeros((64, BCOLS), f32).at[:, 9].set(W2 @ att_dst2[0])

    z1 = jnp.zeros((ROWS_PER_SUB, ACC1_COLS), f32)
    z2 = jnp.zeros((ROWS_PER_SUB, ACC2_COLS), f32)

    a1, b1t = _run_k1(x.astype(f32), wa1, m1, wb1)
    p1 = _sc_pass1(a1, b1t, src, dst, z1)
    a2, b2t = _run_k2(p1, b1.reshape(1, 64), e8, w2a, m2, w2b)
    p2 = _sc_pass2(a2, b2t, src, dst, z2)
    out = _run_k3(p2, b2.reshape(1, NCLS))
    return out[:N_NODES]


# speedup vs baseline: 1.1921x; 1.1921x over previous
import functools

import jax
import jax.numpy as jnp
from jax import lax
from jax.experimental import pallas as pl
from jax.experimental.pallas import tpu as pltpu
from jax.experimental.pallas import tpu_sc as plsc

N_NODES = 10000
N_EDGES = 160000
D_IN = 256
H1, C1 = 8, 8
NCLS = 40

NPAD = 10016
NW = 32
GROUPS = 40
GB = 128
EPAD = NW * GROUPS * GB
ROWS_PER_SUB = NPAD // 16

ACC1_COLS = 72
TBL1_COLS = 96
ACC2_COLS = 48
TBL2_COLS = 64
BCOLS = 16


def _take16(v, idx):
    return lax.gather(
        v, idx[:, None],
        lax.GatherDimensionNumbers(
            offset_dims=(), collapsed_slice_dims=(0,), start_index_map=(0,)),
        slice_sizes=(1,),
        mode=lax.GatherScatterMode.PROMISE_IN_BOUNDS)


def _k1_body(x_ref, wa_ref, m_ref, wb_ref, a_out, b_out):
    xb = x_ref[...]
    a = jnp.dot(xb, wa_ref[...], preferred_element_type=jnp.float32)
    a_out[...] = (a + m_ref[...]).astype(jnp.bfloat16)
    b_out[...] = jnp.dot(xb, wb_ref[...], preferred_element_type=jnp.float32)


def _run_k1(x_pad, wa1, m1, wb1):
    blk = NPAD // 4
    grid = 4
    return pl.pallas_call(
        _k1_body,
        grid=(grid,),
        in_specs=[
            pl.BlockSpec((blk, D_IN), lambda i: (i, 0)),
            pl.BlockSpec((D_IN, TBL1_COLS), lambda i: (0, 0)),
            pl.BlockSpec((1, TBL1_COLS), lambda i: (0, 0)),
            pl.BlockSpec((D_IN, BCOLS), lambda i: (0, 0)),
        ],
        out_specs=[
            pl.BlockSpec((blk, TBL1_COLS), lambda i: (i, 0)),
            pl.BlockSpec((blk, BCOLS), lambda i: (i, 0)),
        ],
        out_shape=[
            jax.ShapeDtypeStruct((NPAD, TBL1_COLS), jnp.bfloat16),
            jax.ShapeDtypeStruct((NPAD, BCOLS), jnp.float32),
        ],
    )(x_pad, wa1, m1, wb1)


def _edge_body1(rows_a, rows_b, ctr, e):
    u0 = rows_a[e, pl.ds(0, 32)].astype(jnp.float32)
    u1 = rows_a[e, pl.ds(32, 32)].astype(jnp.float32)
    u2 = rows_a[e, pl.ds(64, 32)].astype(jnp.float32)
    g0, g1 = u0[0:16], u0[16:32]
    g2, g3 = u1[0:16], u1[16:32]
    asv = u2[0:16]
    a_d = rows_b[e, :]
    al = asv + a_d
    al = jnp.maximum(al, al * jnp.float32(0.2))
    ex = jnp.exp(al)
    iot = lax.iota(jnp.int32, 16)
    half = lax.shift_right_logical(iot, 3)
    low8 = iot < 8
    gs = (g0, g1, g2, g3)
    for k in range(4):
        m = _take16(ex, half + 2 * k)
        ctr[e, pl.ds(16 * k, 16)] = gs[k] * m
    g3s = _take16(g3, jnp.where(low8, iot + 8, 0))
    v = jnp.where(low8, g3s, jnp.float32(1.0))
    mlast = _take16(ex, jnp.where(low8, 7, iot - 8))
    ctr[e, pl.ds(56, 16)] = v * mlast


def _edge_body2(rows_a, rows_b, ctr, e):
    u0 = rows_a[e, pl.ds(0, 32)].astype(jnp.float32)
    u1 = rows_a[e, pl.ds(32, 32)].astype(jnp.float32)
    g0, g1 = u0[0:16], u0[16:32]
    g2 = u1[0:16]
    a_d = rows_b[e, :]
    al = g2 + a_d
    al = jnp.maximum(al, al * jnp.float32(0.2))
    ex = jnp.exp(al)
    m = _take16(ex, jnp.full((16,), 9, jnp.int32))
    gs = (g0, g1, g2)
    for k in range(3):
        ctr[e, pl.ds(16 * k, 16)] = gs[k] * m


def _make_sc_pass(acc_cols, tbl_cols, edge_body, stage_a):
    mesh = plsc.VectorSubcoreMesh(core_axis_name="c", subcore_axis_name="s")

    @functools.partial(
        pl.kernel, mesh=mesh,
        compiler_params=pltpu.CompilerParams(use_tc_tiling_on_sc=False),
        out_type=[jax.ShapeDtypeStruct((NPAD, acc_cols), jnp.float32),
                  jax.ShapeDtypeStruct((NPAD, acc_cols), jnp.float32)],
        scratch_types=[
            pltpu.VMEM_SHARED((NPAD, acc_cols), jnp.float32),
            pltpu.VMEM_SHARED((NPAD if stage_a else 1,
                               tbl_cols), jnp.bfloat16),
            pltpu.VMEM((GROUPS, GB), jnp.int32),
            pltpu.VMEM((GROUPS, GB), jnp.int32),
            pltpu.VMEM((2, GB, tbl_cols), jnp.bfloat16),
            pltpu.VMEM((2, GB, BCOLS), jnp.float32),
            pltpu.VMEM((2, GB, acc_cols), jnp.float32),
            pltpu.SemaphoreType.DMA,
            pltpu.SemaphoreType.DMA,
            pltpu.SemaphoreType.DMA,
            pltpu.SemaphoreType.DMA,
            pltpu.SemaphoreType.DMA,
            pltpu.SemaphoreType.DMA,
        ],
    )
    def sc_pass(a_hbm, b_hbm, src_hbm, dst_hbm, z_hbm, out0_hbm, out1_hbm,
                acc, a_sp, sidx, didx, rows_a, rows_b, ctr,
                sa0, sa1, sb0, sb1, ss0, ss1):
        c = lax.axis_index("c")
        s = lax.axis_index("s")
        w = s * 2 + c
        r0 = s * ROWS_PER_SUB
        sem_a = (sa0, sa1)
        sem_b = (sb0, sb1)
        sem_s = (ss0, ss1)
        rows_slice = pl.ds(r0, ROWS_PER_SUB)
        pltpu.sync_copy(z_hbm, acc.at[rows_slice])
        if stage_a:
            pltpu.sync_copy(a_hbm.at[rows_slice], a_sp.at[rows_slice])
        pltpu.sync_copy(src_hbm.at[w], sidx)
        pltpu.sync_copy(dst_hbm.at[w], didx)
        plsc.subcore_barrier()

        a_tbl = a_sp if stage_a else a_hbm

        def start_gather(g, b):
            pltpu.async_copy(a_tbl.at[sidx.at[g]], rows_a.at[b], sem_a[b])
            pltpu.async_copy(b_hbm.at[didx.at[g]], rows_b.at[b], sem_b[b])

        start_gather(0, 0)
        start_gather(1, 1)

        def gp_body(gp, carry):
            for b in range(2):
                g = 2 * gp + b
                pltpu.make_async_copy(
                    a_tbl.at[sidx.at[g]], rows_a.at[b], sem_a[b]).wait()
                pltpu.make_async_copy(
                    b_hbm.at[didx.at[g]], rows_b.at[b], sem_b[b]).wait()

                @pl.when(gp > 0)
                def _():
                    pltpu.make_async_copy(
                        ctr.at[b], acc.at[didx.at[g]], sem_s[b]).wait()

                @plsc.parallel_loop(0, GB, unroll=8)
                def _(e):
                    edge_body(rows_a.at[b], rows_b.at[b], ctr.at[b], e)

                pltpu.async_copy(ctr.at[b], acc.at[didx.at[g]], sem_s[b],
                                 add=True)

                @pl.when(g + 2 < GROUPS)
                def _():
                    start_gather(g + 2, b)
            return carry
        lax.fori_loop(0, GROUPS // 2, gp_body, 0)

        for b in range(2):
            pltpu.make_async_copy(
                ctr.at[b], acc.at[didx.at[GROUPS - 2 + b]], sem_s[b]).wait()

        plsc.subcore_barrier()

        @pl.when(c == 0)
        def _():
            pltpu.sync_copy(acc.at[rows_slice], out0_hbm.at[rows_slice])

        @pl.when(c == 1)
        def _():
            pltpu.sync_copy(acc.at[rows_slice], out1_hbm.at[rows_slice])

    return sc_pass


def _k2_body(p0_ref, p1_ref, b1_ref, e8_ref, w2a_ref, m2_ref, w2b_ref,
             a_out, b_out):
    acc = p0_ref[...] + p1_ref[...]
    num = acc[:, 0:64]
    den = jnp.dot(acc[:, 64:72], e8_ref[...],
                  preferred_element_type=jnp.float32)
    h1 = num / (den + jnp.float32(1e-16)) + b1_ref[...]
    h1 = jnp.where(h1 > 0, h1, jnp.exp(jnp.minimum(h1, 0.0)) - 1.0)
    a2 = jnp.dot(h1, w2a_ref[...], preferred_element_type=jnp.float32)
    a_out[...] = (a2 + m2_ref[...]).astype(jnp.bfloat16)
    b_out[...] = jnp.dot(h1, w2b_ref[...], preferred_element_type=jnp.float32)


def _run_k2(p1, b1, e8, w2a, m2, w2b):
    blk = NPAD // 4
    grid = 4
    return pl.pallas_call(
        _k2_body,
        grid=(grid,),
        in_specs=[
            pl.BlockSpec((blk, ACC1_COLS), lambda i: (i, 0)),
            pl.BlockSpec((blk, ACC1_COLS), lambda i: (i, 0)),
            pl.BlockSpec((1, 64), lambda i: (0, 0)),
            pl.BlockSpec((8, 64), lambda i: (0, 0)),
            pl.BlockSpec((64, TBL2_COLS), lambda i: (0, 0)),
            pl.BlockSpec((1, TBL2_COLS), lambda i: (0, 0)),
            pl.BlockSpec((64, BCOLS), lambda i: (0, 0)),
        ],
        out_specs=[
            pl.BlockSpec((blk, TBL2_COLS), lambda i: (i, 0)),
            pl.BlockSpec((blk, BCOLS), lambda i: (i, 0)),
        ],
        out_shape=[
            jax.ShapeDtypeStruct((NPAD, TBL2_COLS), jnp.bfloat16),
            jax.ShapeDtypeStruct((NPAD, BCOLS), jnp.float32),
        ],
    )(p1[0], p1[1], b1, e8, w2a, m2, w2b)


def _k3_body(p0_ref, p1_ref, b2_ref, out_ref):
    acc = p0_ref[...] + p1_ref[...]
    num = acc[:, 0:NCLS]
    den = acc[:, NCLS:NCLS + 1]
    logits = num / (den + jnp.float32(1e-16)) + b2_ref[...]
    mx = jnp.max(logits, axis=1, keepdims=True)
    lse = jnp.log(jnp.sum(jnp.exp(logits - mx), axis=1, keepdims=True))
    out_ref[...] = logits - mx - lse


def _run_k3(p2, b2):
    blk = NPAD // 4
    grid = 4
    return pl.pallas_call(
        _k3_body,
        grid=(grid,),
        in_specs=[
            pl.BlockSpec((blk, ACC2_COLS), lambda i: (i, 0)),
            pl.BlockSpec((blk, ACC2_COLS), lambda i: (i, 0)),
            pl.BlockSpec((1, NCLS), lambda i: (0, 0)),
        ],
        out_specs=pl.BlockSpec((blk, NCLS), lambda i: (i, 0)),
        out_shape=jax.ShapeDtypeStruct((NPAD, NCLS), jnp.float32),
    )(p2[0], p2[1], b2)


_sc_pass1 = _make_sc_pass(ACC1_COLS, TBL1_COLS, _edge_body1, stage_a=True)
_sc_pass2 = _make_sc_pass(ACC2_COLS, TBL2_COLS, _edge_body2, stage_a=True)


@jax.jit
def kernel(x, edge_index, W1, att_src1, att_dst1, b1, W2, att_src2, att_dst2, b2):
    f32 = jnp.float32
    ei = edge_index.astype(jnp.int32)
    pad = jnp.full((EPAD - N_EDGES,), N_NODES, jnp.int32)
    src = jnp.concatenate([ei[0], pad]).reshape(NW, GROUPS, GB)
    dst = jnp.concatenate([ei[1], pad]).reshape(NW, GROUPS, GB)

    e8 = jnp.repeat(jnp.eye(H1, dtype=f32), C1, axis=1).reshape(H1, H1 * C1)

    wsrc1 = (W1 * att_src1.reshape(1, H1 * C1)) @ e8.T
    wdst1 = (W1 * att_dst1.reshape(1, H1 * C1)) @ e8.T

    wa1 = jnp.concatenate([W1, wsrc1, jnp.zeros((D_IN, 24), f32)], axis=1)
    m1 = (((jnp.arange(TBL1_COLS) >= 72)
           & (jnp.arange(TBL1_COLS) < 80)).astype(f32)
          .reshape(1, TBL1_COLS))
    wb1 = jnp.concatenate([wdst1, jnp.zeros((D_IN, 8), f32)], axis=1)

    w2a = jnp.concatenate(
        [W2, jnp.zeros((64, 1), f32), (W2 @ att_src2[0])[:, None],
         jnp.zeros((64, 22), f32)], axis=1)
    m2 = (jnp.arange(TBL2_COLS) == 40).astype(f32).reshape(1, TBL2_COLS)
    w2b = jnp.zeros((64, BCOLS), f32).at[:, 9].set(W2 @ att_dst2[0])

    z1 = jnp.zeros((ROWS_PER_SUB, ACC1_COLS), f32)
    z2 = jnp.zeros((ROWS_PER_SUB, ACC2_COLS), f32)

    a1, b1t = _run_k1(x.astype(f32), wa1, m1, wb1)
    p1 = _sc_pass1(a1, b1t, src, dst, z1)
    a2, b2t = _run_k2(p1, b1.reshape(1, 64), e8, w2a, m2, w2b)
    p2 = _sc_pass2(a2, b2t, src, dst, z2)
    out = _run_k3(p2, b2.reshape(1, NCLS))
    return out[:N_NODES]


# speedup vs baseline: 1.2212x; 1.0244x over previous
import functools

import jax
import jax.numpy as jnp
from jax import lax
from jax.experimental import pallas as pl
from jax.experimental.pallas import tpu as pltpu
from jax.experimental.pallas import tpu_sc as plsc

N_NODES = 10000
N_EDGES = 160000
D_IN = 256
H1, C1 = 8, 8
NCLS = 40

NPAD = 10016
NW = 32
GROUPS = 40
GB = 128
EPAD = NW * GROUPS * GB
ROWS_PER_SUB = NPAD // 16

ACC1_COLS = 72
TBL1_COLS = 96
ACC2_COLS = 48
TBL2_COLS = 64
BCOLS = 16


def _take16(v, idx):
    return lax.gather(
        v, idx[:, None],
        lax.GatherDimensionNumbers(
            offset_dims=(), collapsed_slice_dims=(0,), start_index_map=(0,)),
        slice_sizes=(1,),
        mode=lax.GatherScatterMode.PROMISE_IN_BOUNDS)


def _k1_body(x_ref, wa_ref, m_ref, wb_ref, a_out, b_out):
    xb = x_ref[...]
    a = jnp.dot(xb, wa_ref[...], preferred_element_type=jnp.float32)
    a_out[...] = (a + m_ref[...]).astype(jnp.bfloat16)
    b_out[...] = jnp.dot(xb, wb_ref[...], preferred_element_type=jnp.float32)


def _run_k1(x_pad, wa1, m1, wb1):
    blk = NPAD // 4
    grid = 4
    return pl.pallas_call(
        _k1_body,
        grid=(grid,),
        in_specs=[
            pl.BlockSpec((blk, D_IN), lambda i: (i, 0)),
            pl.BlockSpec((D_IN, TBL1_COLS), lambda i: (0, 0)),
            pl.BlockSpec((1, TBL1_COLS), lambda i: (0, 0)),
            pl.BlockSpec((D_IN, BCOLS), lambda i: (0, 0)),
        ],
        out_specs=[
            pl.BlockSpec((blk, TBL1_COLS), lambda i: (i, 0)),
            pl.BlockSpec((blk, BCOLS), lambda i: (i, 0)),
        ],
        out_shape=[
            jax.ShapeDtypeStruct((NPAD, TBL1_COLS), jnp.bfloat16),
            jax.ShapeDtypeStruct((NPAD, BCOLS), jnp.float32),
        ],
    )(x_pad, wa1, m1, wb1)


def _edge_body1(rows_a, rows_b, ctr, e):
    u0 = rows_a[e, pl.ds(0, 32)].astype(jnp.float32)
    u1 = rows_a[e, pl.ds(32, 32)].astype(jnp.float32)
    u2 = rows_a[e, pl.ds(64, 32)].astype(jnp.float32)
    g0, g1 = u0[0:16], u0[16:32]
    g2, g3 = u1[0:16], u1[16:32]
    asv = u2[0:16]
    a_d = rows_b[e, :]
    al = asv + a_d
    al = jnp.maximum(al, al * jnp.float32(0.2))
    ex = jnp.exp(al)
    iot = lax.iota(jnp.int32, 16)
    half = lax.shift_right_logical(iot, 3)
    low8 = iot < 8
    gs = (g0, g1, g2, g3)
    for k in range(4):
        m = _take16(ex, half + 2 * k)
        ctr[e, pl.ds(16 * k, 16)] = gs[k] * m
    g3s = _take16(g3, jnp.where(low8, iot + 8, 0))
    v = jnp.where(low8, g3s, jnp.float32(1.0))
    mlast = _take16(ex, jnp.where(low8, 7, iot - 8))
    ctr[e, pl.ds(56, 16)] = v * mlast


def _edge_body2(rows_a, rows_b, ctr, e):
    u0 = rows_a[e, pl.ds(0, 32)].astype(jnp.float32)
    u1 = rows_a[e, pl.ds(32, 32)].astype(jnp.float32)
    g0, g1 = u0[0:16], u0[16:32]
    g2 = u1[0:16]
    a_d = rows_b[e, :]
    al = g2 + a_d
    al = jnp.maximum(al, al * jnp.float32(0.2))
    ex = jnp.exp(al)
    m = _take16(ex, jnp.full((16,), 9, jnp.int32))
    gs = (g0, g1, g2)
    for k in range(3):
        ctr[e, pl.ds(16 * k, 16)] = gs[k] * m


def _make_sc_pass(acc_cols, tbl_cols, edge_body, stage_a):
    mesh = plsc.VectorSubcoreMesh(core_axis_name="c", subcore_axis_name="s")

    @functools.partial(
        pl.kernel, mesh=mesh,
        compiler_params=pltpu.CompilerParams(use_tc_tiling_on_sc=False),
        out_type=[jax.ShapeDtypeStruct((NPAD, acc_cols), jnp.float32),
                  jax.ShapeDtypeStruct((NPAD, acc_cols), jnp.float32)],
        scratch_types=[
            pltpu.VMEM_SHARED((NPAD, acc_cols), jnp.float32),
            pltpu.VMEM_SHARED((NPAD if stage_a else 1,
                               tbl_cols), jnp.bfloat16),
            pltpu.VMEM((GROUPS, GB), jnp.int32),
            pltpu.VMEM((GROUPS, GB), jnp.int32),
            pltpu.VMEM((2, GB, tbl_cols), jnp.bfloat16),
            pltpu.VMEM((2, GB, BCOLS), jnp.float32),
            pltpu.VMEM((2, GB, acc_cols), jnp.float32),
            pltpu.SemaphoreType.DMA,
            pltpu.SemaphoreType.DMA,
            pltpu.SemaphoreType.DMA,
            pltpu.SemaphoreType.DMA,
            pltpu.SemaphoreType.DMA,
            pltpu.SemaphoreType.DMA,
        ],
    )
    def sc_pass(a_hbm, b_hbm, src_hbm, dst_hbm, z_hbm, out0_hbm, out1_hbm,
                acc, a_sp, sidx, didx, rows_a, rows_b, ctr,
                sa0, sa1, sb0, sb1, ss0, ss1):
        c = lax.axis_index("c")
        s = lax.axis_index("s")
        w = s * 2 + c
        r0 = s * ROWS_PER_SUB
        sem_a = (sa0, sa1)
        sem_b = (sb0, sb1)
        sem_s = (ss0, ss1)
        rows_slice = pl.ds(r0, ROWS_PER_SUB)
        pltpu.sync_copy(z_hbm, acc.at[rows_slice])
        if stage_a:
            pltpu.sync_copy(a_hbm.at[rows_slice], a_sp.at[rows_slice])
        pltpu.sync_copy(src_hbm.at[w], sidx)
        pltpu.sync_copy(dst_hbm.at[w], didx)
        plsc.subcore_barrier()

        a_tbl = a_sp if stage_a else a_hbm

        def start_gather(g, b):
            pltpu.async_copy(a_tbl.at[sidx.at[g]], rows_a.at[b], sem_a[b])
            pltpu.async_copy(b_hbm.at[didx.at[g]], rows_b.at[b], sem_b[b])

        start_gather(0, 0)
        start_gather(1, 1)

        def gp_body(gp, carry):
            for b in range(2):
                g = 2 * gp + b
                pltpu.make_async_copy(
                    a_tbl.at[sidx.at[g]], rows_a.at[b], sem_a[b]).wait()
                pltpu.make_async_copy(
                    b_hbm.at[didx.at[g]], rows_b.at[b], sem_b[b]).wait()

                @pl.when(gp > 0)
                def _():
                    pltpu.make_async_copy(
                        ctr.at[b], acc.at[didx.at[g]], sem_s[b]).wait()

                @plsc.parallel_loop(0, GB, unroll=4)
                def _(e):
                    edge_body(rows_a.at[b], rows_b.at[b], ctr.at[b], e)

                pltpu.async_copy(ctr.at[b], acc.at[didx.at[g]], sem_s[b],
                                 add=True)

                @pl.when(g + 2 < GROUPS)
                def _():
                    start_gather(g + 2, b)
            return carry
        lax.fori_loop(0, GROUPS // 2, gp_body, 0)

        for b in range(2):
            pltpu.make_async_copy(
                ctr.at[b], acc.at[didx.at[GROUPS - 2 + b]], sem_s[b]).wait()

        plsc.subcore_barrier()

        @pl.when(c == 0)
        def _():
            pltpu.sync_copy(acc.at[rows_slice], out0_hbm.at[rows_slice])

        @pl.when(c == 1)
        def _():
            pltpu.sync_copy(acc.at[rows_slice], out1_hbm.at[rows_slice])

    return sc_pass


def _k2_body(p0_ref, p1_ref, b1_ref, e8_ref, w2a_ref, m2_ref, w2b_ref,
             a_out, b_out):
    acc = p0_ref[...] + p1_ref[...]
    num = acc[:, 0:64]
    den = jnp.dot(acc[:, 64:72], e8_ref[...],
                  preferred_element_type=jnp.float32)
    h1 = num / (den + jnp.float32(1e-16)) + b1_ref[...]
    h1 = jnp.where(h1 > 0, h1, jnp.exp(jnp.minimum(h1, 0.0)) - 1.0)
    a2 = jnp.dot(h1, w2a_ref[...], preferred_element_type=jnp.float32)
    a_out[...] = (a2 + m2_ref[...]).astype(jnp.bfloat16)
    b_out[...] = jnp.dot(h1, w2b_ref[...], preferred_element_type=jnp.float32)


def _run_k2(p1, b1, e8, w2a, m2, w2b):
    blk = NPAD // 4
    grid = 4
    return pl.pallas_call(
        _k2_body,
        grid=(grid,),
        in_specs=[
            pl.BlockSpec((blk, ACC1_COLS), lambda i: (i, 0)),
            pl.BlockSpec((blk, ACC1_COLS), lambda i: (i, 0)),
            pl.BlockSpec((1, 64), lambda i: (0, 0)),
            pl.BlockSpec((8, 64), lambda i: (0, 0)),
            pl.BlockSpec((64, TBL2_COLS), lambda i: (0, 0)),
            pl.BlockSpec((1, TBL2_COLS), lambda i: (0, 0)),
            pl.BlockSpec((64, BCOLS), lambda i: (0, 0)),
        ],
        out_specs=[
            pl.BlockSpec((blk, TBL2_COLS), lambda i: (i, 0)),
            pl.BlockSpec((blk, BCOLS), lambda i: (i, 0)),
        ],
        out_shape=[
            jax.ShapeDtypeStruct((NPAD, TBL2_COLS), jnp.bfloat16),
            jax.ShapeDtypeStruct((NPAD, BCOLS), jnp.float32),
        ],
    )(p1[0], p1[1], b1, e8, w2a, m2, w2b)


def _k3_body(p0_ref, p1_ref, b2_ref, out_ref):
    acc = p0_ref[...] + p1_ref[...]
    num = acc[:, 0:NCLS]
    den = acc[:, NCLS:NCLS + 1]
    logits = num / (den + jnp.float32(1e-16)) + b2_ref[...]
    mx = jnp.max(logits, axis=1, keepdims=True)
    lse = jnp.log(jnp.sum(jnp.exp(logits - mx), axis=1, keepdims=True))
    out_ref[...] = logits - mx - lse


def _run_k3(p2, b2):
    blk = NPAD // 4
    grid = 4
    return pl.pallas_call(
        _k3_body,
        grid=(grid,),
        in_specs=[
            pl.BlockSpec((blk, ACC2_COLS), lambda i: (i, 0)),
            pl.BlockSpec((blk, ACC2_COLS), lambda i: (i, 0)),
            pl.BlockSpec((1, NCLS), lambda i: (0, 0)),
        ],
        out_specs=pl.BlockSpec((blk, NCLS), lambda i: (i, 0)),
        out_shape=jax.ShapeDtypeStruct((NPAD, NCLS), jnp.float32),
    )(p2[0], p2[1], b2)


_sc_pass1 = _make_sc_pass(ACC1_COLS, TBL1_COLS, _edge_body1, stage_a=True)
_sc_pass2 = _make_sc_pass(ACC2_COLS, TBL2_COLS, _edge_body2, stage_a=True)


@jax.jit
def kernel(x, edge_index, W1, att_src1, att_dst1, b1, W2, att_src2, att_dst2, b2):
    f32 = jnp.float32
    ei = edge_index.astype(jnp.int32)
    pad = jnp.full((EPAD - N_EDGES,), N_NODES, jnp.int32)
    src = jnp.concatenate([ei[0], pad]).reshape(NW, GROUPS, GB)
    dst = jnp.concatenate([ei[1], pad]).reshape(NW, GROUPS, GB)

    e8 = jnp.repeat(jnp.eye(H1, dtype=f32), C1, axis=1).reshape(H1, H1 * C1)

    wsrc1 = (W1 * att_src1.reshape(1, H1 * C1)) @ e8.T
    wdst1 = (W1 * att_dst1.reshape(1, H1 * C1)) @ e8.T

    wa1 = jnp.concatenate([W1, wsrc1, jnp.zeros((D_IN, 24), f32)], axis=1)
    m1 = (((jnp.arange(TBL1_COLS) >= 72)
           & (jnp.arange(TBL1_COLS) < 80)).astype(f32)
          .reshape(1, TBL1_COLS))
    wb1 = jnp.concatenate([wdst1, jnp.zeros((D_IN, 8), f32)], axis=1)

    w2a = jnp.concatenate(
        [W2, jnp.zeros((64, 1), f32), (W2 @ att_src2[0])[:, None],
         jnp.zeros((64, 22), f32)], axis=1)
    m2 = (jnp.arange(TBL2_COLS) == 40).astype(f32).reshape(1, TBL2_COLS)
    w2b = jnp.zeros((64, BCOLS), f32).at[:, 9].set(W2 @ att_dst2[0])

    z1 = jnp.zeros((ROWS_PER_SUB, ACC1_COLS), f32)
    z2 = jnp.zeros((ROWS_PER_SUB, ACC2_COLS), f32)

    a1, b1t = _run_k1(x.astype(f32), wa1, m1, wb1)
    p1 = _sc_pass1(a1, b1t, src, dst, z1)
    a2, b2t = _run_k2(p1, b1.reshape(1, 64), e8, w2a, m2, w2b)
    p2 = _sc_pass2(a2, b2t, src, dst, z2)
    out = _run_k3(p2, b2.reshape(1, NCLS))
    return out[:N_NODES]
